# baseline (speedup 1.0000x reference)
import functools

import jax
import jax.numpy as jnp
from jax import lax
from jax.experimental import pallas as pl
from jax.experimental.pallas import tpu as pltpu
from jax.experimental.pallas import tpu_sc as plsc

N_NODES = 50000
N_POOLED = 25000
D_FEAT = 256
N_EDGES = 800000
E_FLAT = 2 * N_EDGES

NW = 32
L = 16

P_PAD = 25024
SENT = N_POOLED

E_PER_W = E_FLAT // NW
EC = 2000
NEC = E_PER_W // EC

W_ROWS = 1600
RC = 80
NRC = W_ROWS // RC
NG = RC // L
HB = 64
N_SCAN = P_PAD // L - 1

_mesh = plsc.VectorSubcoreMesh(core_axis_name="c", subcore_axis_name="s")


@functools.partial(
    pl.kernel,
    out_type=(
        jax.ShapeDtypeStruct((E_FLAT,), jnp.int32),
        jax.ShapeDtypeStruct((N_NODES, D_FEAT), jnp.float32),
    ),
    mesh=_mesh,
    compiler_params=pltpu.CompilerParams(needs_layout_passes=False),
    scratch_types=[
        pltpu.VMEM((P_PAD,), jnp.int32),
        pltpu.VMEM((W_ROWS,), jnp.int32),
        [pltpu.VMEM((EC,), jnp.int32)] * 2,
        [pltpu.VMEM((EC,), jnp.int32)] * 2,
        [pltpu.VMEM((HB, D_FEAT), jnp.float32)] * 2,
        [pltpu.VMEM((RC, D_FEAT), jnp.float32)] * 2,
        pltpu.SMEM((2,), jnp.int32),
        pltpu.SemaphoreType.DMA,
        pltpu.SemaphoreType.DMA,
        pltpu.SemaphoreType.DMA,
        pltpu.SemaphoreType.DMA,
        pltpu.SemaphoreType.DMA,
    ],
)
def _unpool_sc(idx_hbm, h_hbm, e_hbm, eo_hbm, newh_hbm,
               idxv, srcv, ebuf, obuf, hstage, outbuf, jmref,
               sem_e, sem_eo, sem_h, sem_f, sem_w):
    wid = lax.axis_index("c") * 16 + lax.axis_index("s")
    iota16 = lax.iota(jnp.int32, L)

    pltpu.sync_copy(idx_hbm, idxv)

    eoff = wid * E_PER_W
    cp_in = {}
    cp_in[0] = pltpu.async_copy(e_hbm.at[pl.ds(eoff, EC)], ebuf[0], sem_e)
    for c in range(NEC):
        b = c & 1
        if c + 1 < NEC:
            cp_in[(c + 1) & 1] = pltpu.async_copy(
                e_hbm.at[pl.ds(eoff + (c + 1) * EC, EC)], ebuf[(c + 1) & 1],
                sem_e)
        cp_in[b].wait()
        if c >= 2:
            pltpu.make_async_copy(
                obuf[b], eo_hbm.at[pl.ds(eoff + (c - 2) * EC, EC)],
                sem_eo).wait()

        @pl.loop(0, EC // L, unroll=8)
        def _gather_edges(i, b=b):
            e = ebuf[b][pl.ds(i * L, L)]
            obuf[b][pl.ds(i * L, L)] = plsc.load_gather(idxv, [e])

        pltpu.async_copy(obuf[b], eo_hbm.at[pl.ds(eoff + c * EC, EC)],
                         sem_eo)
    for c in range(max(NEC - 2, 0), NEC):
        pltpu.make_async_copy(
            obuf[c & 1], eo_hbm.at[pl.ds(eoff + c * EC, EC)], sem_eo).wait()

    n0 = wid * W_ROWS

    @pl.loop(0, W_ROWS // L)
    def _fill_sent(i):
        srcv[pl.ds(i * L, L)] = jnp.full((L,), SENT, jnp.int32)

    @pl.loop(0, N_SCAN, unroll=4)
    def _scan_idx(i):
        a = idxv[pl.ds(i * L, L)]
        nxt = idxv[pl.ds(i * L + 1, L)]
        t = a - n0
        j = iota16 + i * L
        m = (a != nxt) & (t >= 0) & (t < W_ROWS)
        plsc.store_scatter(srcv, [t], j, mask=m)

    JMAX = N_POOLED - HB

    def _src_vec(cidx, g):
        return srcv[pl.ds(cidx * RC + g * L, L)]

    def _slab_base(cidx):
        m = _src_vec(cidx, 0)
        for g in range(1, NG):
            m = jnp.minimum(m, _src_vec(cidx, g))
        jm = lax.reduce_min(m, axes=(0,))
        return pl.multiple_of(jnp.clip(jm & ~7, 0, JMAX), 8)

    def _src_max(cidx):
        m = jnp.full((L,), -1, jnp.int32)
        for g in range(NG):
            sv = _src_vec(cidx, g)
            m = jnp.maximum(m, jnp.where(sv == SENT, -1, sv))
        return lax.reduce_max(m, axes=(0,))

    def _shuffle_group(dst, hst, o, ivec, hit=None, fmask=None):
        @pl.loop(0, 256, unroll=8)
        def _mv(k):
            col = (k & 240) + ((iota16 + k) & 15)
            x = plsc.load_gather(hst, [o, col])
            if hit is not None:
                x = jnp.where(hit, x, jnp.float32(0.0))
            plsc.store_scatter(dst, [ivec, col], x, mask=fmask)

    def _write_desc(ub, base):
        return pltpu.make_async_copy(
            outbuf[ub], newh_hbm.at[pl.ds(pl.multiple_of(base, 8), RC)],
            sem_w)

    @pl.loop(0, NRC // 2)
    def _pair(t):
        for u in range(2):
            cidx = 2 * t + u
            base = n0 + cidx * RC

            @pl.when(base < N_NODES)
            def _stage(u=u, cidx=cidx):
                jm = _slab_base(cidx)
                jmref[u] = jm
                pltpu.async_copy(h_hbm.at[pl.ds(jm, HB)], hstage[u], sem_h)

        for u in range(2):
            cidx = 2 * t + u
            base = n0 + cidx * RC

            @pl.when(base < N_NODES)
            def _chunk(u=u, cidx=cidx, base=base):
                jm = pl.multiple_of(jmref[u], 8)

                @pl.when(t > 0)
                def _drain_prev():
                    _write_desc(u, base - 2 * RC).wait()

                pltpu.make_async_copy(
                    h_hbm.at[pl.ds(jm, HB)], hstage[u], sem_h).wait()

                for g in range(NG):
                    sv = _src_vec(cidx, g)
                    o = jnp.clip(sv - jm, 0, HB - 1)
                    hit = sv != SENT
                    ivec = iota16 + g * L
                    _shuffle_group(outbuf[u], hstage[u], o, ivec, hit=hit)

                mo = _src_max(cidx)

                def _fb_body(jb):
                    jbc = pl.multiple_of(jnp.clip(jb, 0, JMAX), 8)
                    pltpu.async_copy(
                        h_hbm.at[pl.ds(jbc, HB)], hstage[u], sem_f).wait()
                    for g in range(NG):
                        sv = _src_vec(cidx, g)
                        o2 = sv - jbc
                        fmask = (o2 >= 0) & (o2 < HB) & (sv != SENT)
                        o2c = jnp.clip(o2, 0, HB - 1)
                        ivec = iota16 + g * L
                        _shuffle_group(outbuf[u], hstage[u], o2c, ivec,
                                       fmask=fmask)
                    return jbc + HB

                lax.while_loop(lambda jb: jb <= mo, _fb_body, jm + HB)

                _write_desc(u, base).start()

    for u in range(2):
        cidx = NRC - 2 + u
        base = n0 + cidx * RC

        @pl.when(base < N_NODES)
        def _drain(u=u, base=base):
            _write_desc(u, base).wait()


def kernel(g, h, idx, edge_index):
    del g
    idx32 = idx.astype(jnp.int32)
    idx_pad = jnp.concatenate(
        [idx32, jnp.full((P_PAD - N_POOLED,), jnp.iinfo(jnp.int32).max,
                         jnp.int32)])
    e_flat = edge_index.astype(jnp.int32).reshape(E_FLAT)
    eo_flat, new_h = _unpool_sc(idx_pad, h.astype(jnp.float32), e_flat)
    return (eo_flat.reshape(2, N_EDGES), new_h)


# speedup vs baseline: 1.8219x; 1.8219x over previous
import functools

import jax
import jax.numpy as jnp
from jax import lax
from jax.experimental import pallas as pl
from jax.experimental.pallas import tpu as pltpu
from jax.experimental.pallas import tpu_sc as plsc

N_NODES = 50000
N_POOLED = 25000
D_FEAT = 256
N_EDGES = 800000
E_FLAT = 2 * N_EDGES

NW = 32
L = 16

P_PAD = 25024
SENT = N_POOLED

E_PER_W = E_FLAT // NW
EC = 2000
NEC = E_PER_W // EC

W_ROWS = 1600
RC = 80
NRC = W_ROWS // RC
NG = RC // L
HB = 64
N_SCAN = P_PAD // L - 1

_mesh = plsc.VectorSubcoreMesh(core_axis_name="c", subcore_axis_name="s")


@functools.partial(
    pl.kernel,
    out_type=(
        jax.ShapeDtypeStruct((E_FLAT,), jnp.int32),
        jax.ShapeDtypeStruct((N_NODES, D_FEAT), jnp.float32),
    ),
    mesh=_mesh,
    compiler_params=pltpu.CompilerParams(needs_layout_passes=False),
    scratch_types=[
        pltpu.VMEM((P_PAD,), jnp.int32),
        pltpu.VMEM((W_ROWS,), jnp.int32),
        [pltpu.VMEM((EC,), jnp.int32)] * 2,
        [pltpu.VMEM((EC,), jnp.int32)] * 2,
        [pltpu.VMEM((HB, D_FEAT), jnp.float32)] * 2,
        [pltpu.VMEM((RC, D_FEAT), jnp.float32)] * 2,
        pltpu.SMEM((2,), jnp.int32),
        pltpu.SemaphoreType.DMA,
        pltpu.SemaphoreType.DMA,
        pltpu.SemaphoreType.DMA,
        pltpu.SemaphoreType.DMA,
        pltpu.SemaphoreType.DMA,
    ],
)
def _unpool_sc(idx_hbm, h_hbm, e_hbm, eo_hbm, newh_hbm,
               idxv, srcv, ebuf, obuf, hstage, outbuf, jmref,
               sem_e, sem_eo, sem_h, sem_f, sem_w):
    wid = lax.axis_index("c") * 16 + lax.axis_index("s")
    iota16 = lax.iota(jnp.int32, L)

    pltpu.sync_copy(idx_hbm, idxv)

    eoff = wid * E_PER_W
    cp_in = {}
    cp_in[0] = pltpu.async_copy(e_hbm.at[pl.ds(eoff, EC)], ebuf[0], sem_e)
    for c in range(NEC):
        b = c & 1
        if c + 1 < NEC:
            cp_in[(c + 1) & 1] = pltpu.async_copy(
                e_hbm.at[pl.ds(eoff + (c + 1) * EC, EC)], ebuf[(c + 1) & 1],
                sem_e)
        cp_in[b].wait()
        if c >= 2:
            pltpu.make_async_copy(
                obuf[b], eo_hbm.at[pl.ds(eoff + (c - 2) * EC, EC)],
                sem_eo).wait()

        @plsc.parallel_loop(0, EC // L, unroll=8)
        def _gather_edges(i, b=b):
            e = ebuf[b][pl.ds(i * L, L)]
            obuf[b][pl.ds(i * L, L)] = plsc.load_gather(idxv, [e])

        pltpu.async_copy(obuf[b], eo_hbm.at[pl.ds(eoff + c * EC, EC)],
                         sem_eo)
    for c in range(max(NEC - 2, 0), NEC):
        pltpu.make_async_copy(
            obuf[c & 1], eo_hbm.at[pl.ds(eoff + c * EC, EC)], sem_eo).wait()

    n0 = wid * W_ROWS

    @pl.loop(0, W_ROWS // L)
    def _fill_sent(i):
        srcv[pl.ds(i * L, L)] = jnp.full((L,), SENT, jnp.int32)

    @pl.loop(0, N_SCAN, unroll=4)
    def _scan_idx(i):
        a = idxv[pl.ds(i * L, L)]
        nxt = idxv[pl.ds(i * L + 1, L)]
        t = a - n0
        j = iota16 + i * L
        m = (a != nxt) & (t >= 0) & (t < W_ROWS)
        plsc.store_scatter(srcv, [t], j, mask=m)

    JMAX = N_POOLED - HB

    def _src_vec(cidx, g):
        return srcv[pl.ds(cidx * RC + g * L, L)]

    def _slab_base(cidx):
        m = _src_vec(cidx, 0)
        for g in range(1, NG):
            m = jnp.minimum(m, _src_vec(cidx, g))
        jm = lax.reduce_min(m, axes=(0,))
        return pl.multiple_of(jnp.clip(jm & ~7, 0, JMAX), 8)

    def _src_max(cidx):
        m = jnp.full((L,), -1, jnp.int32)
        for g in range(NG):
            sv = _src_vec(cidx, g)
            m = jnp.maximum(m, jnp.where(sv == SENT, -1, sv))
        return lax.reduce_max(m, axes=(0,))

    def _shuffle_group(dst, hst, o, ivec, hit=None, fmask=None):
        @plsc.parallel_loop(0, 256, unroll=8)
        def _mv(k):
            col = (k & 240) + ((iota16 + k) & 15)
            x = plsc.load_gather(hst, [o, col])
            if hit is not None:
                x = jnp.where(hit, x, jnp.float32(0.0))
            plsc.store_scatter(dst, [ivec, col], x, mask=fmask)

    def _write_desc(ub, base):
        return pltpu.make_async_copy(
            outbuf[ub], newh_hbm.at[pl.ds(pl.multiple_of(base, 8), RC)],
            sem_w)

    @pl.loop(0, NRC // 2)
    def _pair(t):
        for u in range(2):
            cidx = 2 * t + u
            base = n0 + cidx * RC

            @pl.when(base < N_NODES)
            def _stage(u=u, cidx=cidx):
                jm = _slab_base(cidx)
                jmref[u] = jm
                pltpu.async_copy(h_hbm.at[pl.ds(jm, HB)], hstage[u], sem_h)

        for u in range(2):
            cidx = 2 * t + u
            base = n0 + cidx * RC

            @pl.when(base < N_NODES)
            def _chunk(u=u, cidx=cidx, base=base):
                jm = pl.multiple_of(jmref[u], 8)

                @pl.when(t > 0)
                def _drain_prev():
                    _write_desc(u, base - 2 * RC).wait()

                pltpu.make_async_copy(
                    h_hbm.at[pl.ds(jm, HB)], hstage[u], sem_h).wait()

                for g in range(NG):
                    sv = _src_vec(cidx, g)
                    o = jnp.clip(sv - jm, 0, HB - 1)
                    hit = sv != SENT
                    ivec = iota16 + g * L
                    _shuffle_group(outbuf[u], hstage[u], o, ivec, hit=hit)

                mo = _src_max(cidx)

                def _fb_body(jb):
                    jbc = pl.multiple_of(jnp.clip(jb, 0, JMAX), 8)
                    pltpu.async_copy(
                        h_hbm.at[pl.ds(jbc, HB)], hstage[u], sem_f).wait()
                    for g in range(NG):
                        sv = _src_vec(cidx, g)
                        o2 = sv - jbc
                        fmask = (o2 >= 0) & (o2 < HB) & (sv != SENT)
                        o2c = jnp.clip(o2, 0, HB - 1)
                        ivec = iota16 + g * L
                        _shuffle_group(outbuf[u], hstage[u], o2c, ivec,
                                       fmask=fmask)
                    return jbc + HB

                lax.while_loop(lambda jb: jb <= mo, _fb_body, jm + HB)

                _write_desc(u, base).start()

    for u in range(2):
        cidx = NRC - 2 + u
        base = n0 + cidx * RC

        @pl.when(base < N_NODES)
        def _drain(u=u, base=base):
            _write_desc(u, base).wait()


def kernel(g, h, idx, edge_index):
    del g
    idx32 = idx.astype(jnp.int32)
    idx_pad = jnp.concatenate(
        [idx32, jnp.full((P_PAD - N_POOLED,), jnp.iinfo(jnp.int32).max,
                         jnp.int32)])
    e_flat = edge_index.astype(jnp.int32).reshape(E_FLAT)
    eo_flat, new_h = _unpool_sc(idx_pad, h.astype(jnp.float32), e_flat)
    return (eo_flat.reshape(2, N_EDGES), new_h)


# speedup vs baseline: 1.9953x; 1.0952x over previous
import functools

import jax
import jax.numpy as jnp
from jax import lax
from jax.experimental import pallas as pl
from jax.experimental.pallas import tpu as pltpu
from jax.experimental.pallas import tpu_sc as plsc

N_NODES = 50000
N_POOLED = 25000
D_FEAT = 256
N_EDGES = 800000
E_FLAT = 2 * N_EDGES

NW = 32
L = 16

P_PAD = 25024
SENT = N_POOLED

E_PER_W = E_FLAT // NW
EC = 2000
NEC = E_PER_W // EC

W_ROWS = 1600
RC = 80
NRC = W_ROWS // RC
NG = RC // L
HB = 64
N_SCAN = P_PAD // L - 1

_mesh = plsc.VectorSubcoreMesh(core_axis_name="c", subcore_axis_name="s")


@functools.partial(
    pl.kernel,
    out_type=(
        jax.ShapeDtypeStruct((E_FLAT,), jnp.int32),
        jax.ShapeDtypeStruct((N_NODES, D_FEAT), jnp.float32),
    ),
    mesh=_mesh,
    compiler_params=pltpu.CompilerParams(needs_layout_passes=False),
    scratch_types=[
        pltpu.VMEM((P_PAD,), jnp.int32),
        pltpu.VMEM((W_ROWS,), jnp.int32),
        [pltpu.VMEM((EC,), jnp.int32)] * 2,
        [pltpu.VMEM((EC,), jnp.int32)] * 2,
        [pltpu.VMEM((HB, D_FEAT), jnp.float32)] * 2,
        [pltpu.VMEM((RC, D_FEAT), jnp.float32)] * 2,
        pltpu.SMEM((2,), jnp.int32),
        pltpu.SemaphoreType.DMA,
        pltpu.SemaphoreType.DMA,
        pltpu.SemaphoreType.DMA,
        pltpu.SemaphoreType.DMA,
        pltpu.SemaphoreType.DMA,
    ],
)
def _unpool_sc(idx_hbm, h_hbm, e_hbm, eo_hbm, newh_hbm,
               idxv, srcv, ebuf, obuf, hstage, outbuf, jmref,
               sem_e, sem_eo, sem_h, sem_f, sem_w):
    wid = lax.axis_index("c") * 16 + lax.axis_index("s")
    iota16 = lax.iota(jnp.int32, L)

    pltpu.sync_copy(idx_hbm, idxv)

    eoff = wid * E_PER_W
    cp_in = {}
    cp_in[0] = pltpu.async_copy(e_hbm.at[pl.ds(eoff, EC)], ebuf[0], sem_e)
    for c in range(NEC):
        b = c & 1
        if c + 1 < NEC:
            cp_in[(c + 1) & 1] = pltpu.async_copy(
                e_hbm.at[pl.ds(eoff + (c + 1) * EC, EC)], ebuf[(c + 1) & 1],
                sem_e)
        cp_in[b].wait()
        if c >= 2:
            pltpu.make_async_copy(
                obuf[b], eo_hbm.at[pl.ds(eoff + (c - 2) * EC, EC)],
                sem_eo).wait()

        @plsc.parallel_loop(0, EC // L, unroll=8)
        def _gather_edges(i, b=b):
            e = ebuf[b][pl.ds(i * L, L)]
            obuf[b][pl.ds(i * L, L)] = plsc.load_gather(idxv, [e])

        pltpu.async_copy(obuf[b], eo_hbm.at[pl.ds(eoff + c * EC, EC)],
                         sem_eo)
    for c in range(max(NEC - 2, 0), NEC):
        pltpu.make_async_copy(
            obuf[c & 1], eo_hbm.at[pl.ds(eoff + c * EC, EC)], sem_eo).wait()

    n0 = wid * W_ROWS

    @plsc.parallel_loop(0, W_ROWS // L, unroll=8)
    def _fill_sent(i):
        srcv[pl.ds(i * L, L)] = jnp.full((L,), SENT, jnp.int32)

    @plsc.parallel_loop(0, N_SCAN, unroll=8)
    def _scan_idx(i):
        a = idxv[pl.ds(i * L, L)]
        nxt = idxv[pl.ds(i * L + 1, L)]
        t = a - n0
        j = iota16 + i * L
        m = (a != nxt) & (t >= 0) & (t < W_ROWS)
        plsc.store_scatter(srcv, [t], j, mask=m)

    JMAX = N_POOLED - HB

    def _src_vec(cidx, g):
        return srcv[pl.ds(cidx * RC + g * L, L)]

    def _slab_base(cidx):
        m = _src_vec(cidx, 0)
        for g in range(1, NG):
            m = jnp.minimum(m, _src_vec(cidx, g))
        jm = lax.reduce_min(m, axes=(0,))
        return pl.multiple_of(jnp.clip(jm & ~7, 0, JMAX), 8)

    def _src_max(cidx):
        m = jnp.full((L,), -1, jnp.int32)
        for g in range(NG):
            sv = _src_vec(cidx, g)
            m = jnp.maximum(m, jnp.where(sv == SENT, -1, sv))
        return lax.reduce_max(m, axes=(0,))

    def _shuffle_group(dst, hst, o, ivec, hit=None, fmask=None):
        @plsc.parallel_loop(0, 256, unroll=8)
        def _mv(k):
            col = (k & 240) + ((iota16 + k) & 15)
            x = plsc.load_gather(hst, [o, col])
            if hit is not None:
                x = jnp.where(hit, x, jnp.float32(0.0))
            plsc.store_scatter(dst, [ivec, col], x, mask=fmask)

    def _write_desc(ub, base):
        return pltpu.make_async_copy(
            outbuf[ub], newh_hbm.at[pl.ds(pl.multiple_of(base, 8), RC)],
            sem_w)

    @pl.loop(0, NRC // 2)
    def _pair(t):
        for u in range(2):
            cidx = 2 * t + u
            base = n0 + cidx * RC

            @pl.when(base < N_NODES)
            def _stage(u=u, cidx=cidx):
                jm = _slab_base(cidx)
                jmref[u] = jm
                pltpu.async_copy(h_hbm.at[pl.ds(jm, HB)], hstage[u], sem_h)

        for u in range(2):
            cidx = 2 * t + u
            base = n0 + cidx * RC

            @pl.when(base < N_NODES)
            def _chunk(u=u, cidx=cidx, base=base):
                jm = pl.multiple_of(jmref[u], 8)

                @pl.when(t > 0)
                def _drain_prev():
                    _write_desc(u, base - 2 * RC).wait()

                pltpu.make_async_copy(
                    h_hbm.at[pl.ds(jm, HB)], hstage[u], sem_h).wait()

                for g in range(NG):
                    sv = _src_vec(cidx, g)
                    o = jnp.clip(sv - jm, 0, HB - 1)
                    hit = sv != SENT
                    ivec = iota16 + g * L
                    _shuffle_group(outbuf[u], hstage[u], o, ivec, hit=hit)

                mo = _src_max(cidx)

                def _fb_body(jb):
                    jbc = pl.multiple_of(jnp.clip(jb, 0, JMAX), 8)
                    pltpu.async_copy(
                        h_hbm.at[pl.ds(jbc, HB)], hstage[u], sem_f).wait()
                    for g in range(NG):
                        sv = _src_vec(cidx, g)
                        o2 = sv - jbc
                        fmask = (o2 >= 0) & (o2 < HB) & (sv != SENT)
                        o2c = jnp.clip(o2, 0, HB - 1)
                        ivec = iota16 + g * L
                        _shuffle_group(outbuf[u], hstage[u], o2c, ivec,
                                       fmask=fmask)
                    return jbc + HB

                lax.while_loop(lambda jb: jb <= mo, _fb_body, jm + HB)

                _write_desc(u, base).start()

    for u in range(2):
        cidx = NRC - 2 + u
        base = n0 + cidx * RC

        @pl.when(base < N_NODES)
        def _drain(u=u, base=base):
            _write_desc(u, base).wait()


def kernel(g, h, idx, edge_index):
    del g
    idx32 = idx.astype(jnp.int32)
    idx_pad = jnp.concatenate(
        [idx32, jnp.full((P_PAD - N_POOLED,), jnp.iinfo(jnp.int32).max,
                         jnp.int32)])
    e_flat = edge_index.astype(jnp.int32).reshape(E_FLAT)
    eo_flat, new_h = _unpool_sc(idx_pad, h.astype(jnp.float32), e_flat)
    return (eo_flat.reshape(2, N_EDGES), new_h)
